# Initial kernel scaffold; baseline (speedup 1.0000x reference)
import jax
import jax.numpy as jnp
from jax.experimental import pallas as pl


def kernel(inputs, emb, Wq, Wx, Wr, b, cache_keys, cache_values, Wt, bt):
    raise NotImplementedError("write your pallas kernel here")


# speedup vs baseline: 3.7028x; 3.7028x over previous
import functools

import jax
import jax.numpy as jnp
from jax import lax
from jax.experimental import pallas as pl
from jax.experimental.pallas import tpu as pltpu
from jax.experimental.pallas import tpu_sc as plsc

NUM_STEPS = 8
BATCH = 256
VOCAB = 10000
EMB = 512
HID = 1024
N_CACHE = 8192
DK = 256
DV = 256
TOP_K = 8

ROWS = NUM_STEPS * BATCH


def _emb_gather(emb, ids):
    info = plsc.get_sparse_core_info()
    nc = info.num_cores
    nw = nc * info.num_subcores
    b_per_w = ROWS // nw
    mesh = plsc.VectorSubcoreMesh(core_axis_name="c", subcore_axis_name="s")

    @functools.partial(
        pl.kernel,
        mesh=mesh,
        out_type=jax.ShapeDtypeStruct((ROWS, EMB), jnp.float32),
        scratch_types=[
            pltpu.VMEM((b_per_w,), jnp.int32),
            pltpu.VMEM((b_per_w, EMB), jnp.float32),
            pltpu.SemaphoreType.DMA,
        ],
    )
    def gather_kernel(table_hbm, idx_hbm, out_hbm, idx_v, rows_v, sem):
        wid = lax.axis_index("s") * nc + lax.axis_index("c")
        base = wid * b_per_w
        pltpu.sync_copy(idx_hbm.at[pl.ds(base, b_per_w)], idx_v)
        pltpu.async_copy(table_hbm.at[idx_v], rows_v, sem).wait()
        pltpu.sync_copy(rows_v, out_hbm.at[pl.ds(base, b_per_w)])

    return gather_kernel(emb, ids)


def _retrieval_body(x_ref, wq_ref, ck_ref, cv_ref, wx_ref, wr_ref, b_ref, h_ref):
    x = x_ref[...]
    q = jnp.dot(x, wq_ref[...])
    scores = lax.dot_general(q, ck_ref[...], (((1,), (1,)), ((), ())))
    col = lax.broadcasted_iota(jnp.int32, scores.shape, 1)
    neg_inf = jnp.float32(-jnp.inf)
    t = scores
    for _ in range(TOP_K):
        m = jnp.max(t, axis=1, keepdims=True)
        eq = t == m
        first = jnp.min(jnp.where(eq, col, jnp.int32(N_CACHE)), axis=1, keepdims=True)
        hit = col == first
        t = jnp.where(hit, neg_inf, t)
    sel = t == neg_inf
    m0 = jnp.max(scores, axis=1, keepdims=True)
    e = jnp.where(sel, jnp.exp(scores - m0), jnp.float32(0.0))
    w = e / jnp.sum(e, axis=1, keepdims=True)
    read = jnp.dot(w, cv_ref[...])
    h_ref[...] = jnp.tanh(
        jnp.dot(x, wx_ref[...]) + jnp.dot(read, wr_ref[...]) + b_ref[...]
    )


def _retrieval(x, Wq, ck, cv, Wx, Wr, b):
    return pl.pallas_call(
        _retrieval_body,
        grid=(NUM_STEPS,),
        in_specs=[
            pl.BlockSpec((BATCH, EMB), lambda i: (i, 0)),
            pl.BlockSpec((EMB, DK), lambda i: (0, 0)),
            pl.BlockSpec((N_CACHE, DK), lambda i: (0, 0)),
            pl.BlockSpec((N_CACHE, DV), lambda i: (0, 0)),
            pl.BlockSpec((EMB, HID), lambda i: (0, 0)),
            pl.BlockSpec((DV, HID), lambda i: (0, 0)),
            pl.BlockSpec((1, HID), lambda i: (0, 0)),
        ],
        out_specs=pl.BlockSpec((BATCH, HID), lambda i: (i, 0)),
        out_shape=jax.ShapeDtypeStruct((ROWS, HID), jnp.float32),
    )(x, Wq, ck, cv, Wx, Wr, b.reshape(1, HID))


def _logits_body(h_ref, wt_ref, bt_ref, out_ref):
    out_ref[...] = jnp.dot(h_ref[...], wt_ref[...]) + bt_ref[...]


def _logits(h, Wt, bt):
    VB = 1024
    return pl.pallas_call(
        _logits_body,
        grid=(pl.cdiv(VOCAB, VB),),
        in_specs=[
            pl.BlockSpec((ROWS, HID), lambda j: (0, 0)),
            pl.BlockSpec((HID, VB), lambda j: (0, j)),
            pl.BlockSpec((1, VB), lambda j: (0, j)),
        ],
        out_specs=pl.BlockSpec((ROWS, VB), lambda j: (0, j)),
        out_shape=jax.ShapeDtypeStruct((ROWS, VOCAB), jnp.float32),
    )(h, Wt, bt.reshape(1, VOCAB))


def kernel(inputs, emb, Wq, Wx, Wr, b, cache_keys, cache_values, Wt, bt):
    ids = jnp.asarray(inputs, jnp.int32).reshape(ROWS)
    x = _emb_gather(emb, ids)
    h = _retrieval(x, Wq, cache_keys, cache_values, Wx, Wr, b)
    logits = _logits(h, Wt, bt)
    return logits.reshape(NUM_STEPS, BATCH, VOCAB)


# speedup vs baseline: 6.9576x; 1.8790x over previous
import functools

import jax
import jax.numpy as jnp
from jax import lax
from jax.experimental import pallas as pl
from jax.experimental.pallas import tpu as pltpu
from jax.experimental.pallas import tpu_sc as plsc

NUM_STEPS = 8
BATCH = 256
VOCAB = 10000
EMB = 512
HID = 1024
N_CACHE = 8192
DK = 256
DV = 256
TOP_K = 8

ROWS = NUM_STEPS * BATCH


def _emb_gather(emb, ids):
    info = plsc.get_sparse_core_info()
    nc = info.num_cores
    nw = nc * info.num_subcores
    b_per_w = ROWS // nw
    mesh = plsc.VectorSubcoreMesh(core_axis_name="c", subcore_axis_name="s")

    @functools.partial(
        pl.kernel,
        mesh=mesh,
        out_type=jax.ShapeDtypeStruct((ROWS, EMB), jnp.float32),
        scratch_types=[
            pltpu.VMEM((b_per_w,), jnp.int32),
            pltpu.VMEM((b_per_w, EMB), jnp.float32),
            pltpu.SemaphoreType.DMA,
        ],
    )
    def gather_kernel(table_hbm, idx_hbm, out_hbm, idx_v, rows_v, sem):
        wid = lax.axis_index("s") * nc + lax.axis_index("c")
        base = wid * b_per_w
        pltpu.sync_copy(idx_hbm.at[pl.ds(base, b_per_w)], idx_v)
        pltpu.async_copy(table_hbm.at[idx_v], rows_v, sem).wait()
        pltpu.sync_copy(rows_v, out_hbm.at[pl.ds(base, b_per_w)])

    return gather_kernel(emb, ids)


def _retrieval_body(x_ref, wq_ref, ck_ref, cv_ref, wx_ref, wr_ref, b_ref, h_ref):
    x = x_ref[...]
    q = jnp.dot(x, wq_ref[...])
    scores = lax.dot_general(q, ck_ref[...], (((1,), (1,)), ((), ())))
    neg_inf = jnp.float32(-jnp.inf)
    t = scores
    m0 = None
    for _ in range(TOP_K):
        m = jnp.max(t, axis=1, keepdims=True)
        if m0 is None:
            m0 = m
        t = jnp.where(t == m, neg_inf, t)
    sel = t == neg_inf
    e = jnp.where(sel, jnp.exp(scores - m0), jnp.float32(0.0))
    z = jnp.sum(e, axis=1, keepdims=True)
    read = jnp.dot(e, cv_ref[...]) / z
    h_ref[...] = jnp.tanh(
        jnp.dot(x, wx_ref[...]) + jnp.dot(read, wr_ref[...]) + b_ref[...]
    )


def _retrieval(x, Wq, ck, cv, Wx, Wr, b):
    return pl.pallas_call(
        _retrieval_body,
        grid=(NUM_STEPS,),
        in_specs=[
            pl.BlockSpec((BATCH, EMB), lambda i: (i, 0)),
            pl.BlockSpec((EMB, DK), lambda i: (0, 0)),
            pl.BlockSpec((N_CACHE, DK), lambda i: (0, 0)),
            pl.BlockSpec((N_CACHE, DV), lambda i: (0, 0)),
            pl.BlockSpec((EMB, HID), lambda i: (0, 0)),
            pl.BlockSpec((DV, HID), lambda i: (0, 0)),
            pl.BlockSpec((1, HID), lambda i: (0, 0)),
        ],
        out_specs=pl.BlockSpec((BATCH, HID), lambda i: (i, 0)),
        out_shape=jax.ShapeDtypeStruct((ROWS, HID), jnp.float32),
    )(x, Wq, ck, cv, Wx, Wr, b.reshape(1, HID))


def _logits_body(h_ref, wtT_ref, btT_ref, out_ref):
    res = lax.dot_general(wtT_ref[...], h_ref[...], (((1,), (1,)), ((), ())))
    res = res + btT_ref[...]
    for s in range(NUM_STEPS):
        out_ref[s] = res[:, s * BATCH:(s + 1) * BATCH]


def _logits(h, Wt, bt):
    VB = 1024
    return pl.pallas_call(
        _logits_body,
        grid=(pl.cdiv(VOCAB, VB),),
        in_specs=[
            pl.BlockSpec((ROWS, HID), lambda j: (0, 0)),
            pl.BlockSpec((VB, HID), lambda j: (j, 0)),
            pl.BlockSpec((VB, 1), lambda j: (j, 0)),
        ],
        out_specs=pl.BlockSpec((NUM_STEPS, VB, BATCH), lambda j: (0, j, 0)),
        out_shape=jax.ShapeDtypeStruct((NUM_STEPS, VOCAB, BATCH), jnp.float32),
    )(h, Wt.T, bt.reshape(VOCAB, 1))


def kernel(inputs, emb, Wq, Wx, Wr, b, cache_keys, cache_values, Wt, bt):
    ids = jnp.asarray(inputs, jnp.int32).reshape(ROWS)
    x = _emb_gather(emb, ids)
    h = _retrieval(x, Wq, cache_keys, cache_values, Wx, Wr, b)
    logits = _logits(h, Wt, bt)
    return jnp.transpose(logits, (0, 2, 1))
